# Initial kernel scaffold; baseline (speedup 1.0000x reference)
import functools

import jax
import jax.numpy as jnp
from jax import lax
from jax.experimental import pallas as pl
from jax.experimental.pallas import tpu as pltpu
from jax.experimental.pallas import tpu_sc as plsc

N = 10000
D = 128
E = 320000
LBL = 200000
NPAD = 10240

NC = 2
NS = 16
LANES = 16

CHUNK = 400
E_CHUNKS = E // CHUNK
E_CHUNKS_PER_TILE = E_CHUNKS // (NC * NS)
L_CHUNKS = LBL // CHUNK

_mesh = plsc.VectorSubcoreMesh(core_axis_name="c", subcore_axis_name="s")


def _fill(ref, start, count, value):
    v = jnp.full((LANES,), value, ref.dtype)

    def body(i, _):
        ref[pl.ds(start + i * LANES, LANES)] = v
        return 0

    lax.fori_loop(0, count // LANES, body, 0)


@functools.partial(
    pl.kernel,
    out_type=jax.ShapeDtypeStruct((NPAD,), jnp.float32),
    mesh=_mesh,
    scratch_types=[
        pltpu.VMEM((CHUNK,), jnp.int32),
        pltpu.VMEM((CHUNK,), jnp.float32),
        pltpu.VMEM((NPAD // NS,), jnp.float32),
        pltpu.VMEM_SHARED((NPAD,), jnp.float32),
    ],
)
def _deg_kernel(dst_hbm, dinv_hbm, idx_v, ones_v, slice_v, deg_sh):
    cid = lax.axis_index("c")
    sid = lax.axis_index("s")
    per = NPAD // NS

    _fill(slice_v, 0, per, 0.0)
    pltpu.sync_copy(slice_v, deg_sh.at[pl.ds(sid * per, per)])
    _fill(ones_v, 0, CHUNK, 1.0)
    plsc.subcore_barrier()

    def chunk_body(t, _):
        c = sid * (E_CHUNKS // NS) + t
        pltpu.sync_copy(dst_hbm.at[pl.ds(c * CHUNK, CHUNK)], idx_v)
        pltpu.sync_copy(ones_v, deg_sh.at[idx_v], add=True)
        return 0

    lax.fori_loop(0, E_CHUNKS // NS, chunk_body, 0)
    plsc.subcore_barrier()

    pltpu.sync_copy(deg_sh.at[pl.ds(sid * per, per)], slice_v)

    def rsqrt_body(i, _):
        d = slice_v[pl.ds(i * LANES, LANES)] + 1.0
        bits = plsc.bitcast(d, jnp.int32)
        y = plsc.bitcast(jnp.int32(0x5F3759DF) - (bits >> 1), jnp.float32)
        half = 0.5 * d
        y = y * (1.5 - half * y * y)
        y = y * (1.5 - half * y * y)
        y = y * (1.5 - half * y * y)
        slice_v[pl.ds(i * LANES, LANES)] = y
        return 0

    lax.fori_loop(0, per // LANES, rsqrt_body, 0)

    @pl.when(cid == 0)
    def _():
        pltpu.sync_copy(slice_v, dinv_hbm.at[pl.ds(sid * per, per)])


@functools.partial(
    pl.kernel,
    out_type=jax.ShapeDtypeStruct((NC, N, D), jnp.float32),
    mesh=_mesh,
    scratch_types=[
        pltpu.VMEM((CHUNK,), jnp.int32),
        pltpu.VMEM((CHUNK,), jnp.int32),
        pltpu.VMEM((CHUNK, D), jnp.float32),
        pltpu.VMEM_SHARED((N, D), jnp.float32),
    ],
)
def _agg_kernel(g_hbm, src_hbm, dst_hbm, out_hbm, src_v, dst_v, rows_v, acc_sh):
    cid = lax.axis_index("c")
    sid = lax.axis_index("s")
    rows_per_tile = N // NS

    pltpu.sync_copy(
        g_hbm.at[pl.ds(sid * rows_per_tile, rows_per_tile)],
        acc_sh.at[pl.ds(sid * rows_per_tile, rows_per_tile)],
    )
    plsc.subcore_barrier()

    wid = sid * NC + cid

    def chunk_body(t, _):
        c = wid * E_CHUNKS_PER_TILE + t
        pltpu.sync_copy(src_hbm.at[pl.ds(c * CHUNK, CHUNK)], src_v)
        pltpu.sync_copy(dst_hbm.at[pl.ds(c * CHUNK, CHUNK)], dst_v)
        pltpu.sync_copy(g_hbm.at[src_v], rows_v)
        pltpu.sync_copy(rows_v, acc_sh.at[dst_v], add=True)
        return 0

    lax.fori_loop(0, E_CHUNKS_PER_TILE, chunk_body, 0)
    plsc.subcore_barrier()

    pltpu.sync_copy(
        acc_sh.at[pl.ds(sid * rows_per_tile, rows_per_tile)],
        out_hbm.at[cid, pl.ds(sid * rows_per_tile, rows_per_tile)],
    )


@functools.partial(
    pl.kernel,
    out_type=jax.ShapeDtypeStruct((LBL,), jnp.float32),
    mesh=_mesh,
    scratch_types=[
        pltpu.VMEM((CHUNK,), jnp.int32),
        pltpu.VMEM((CHUNK,), jnp.int32),
        pltpu.VMEM((CHUNK, D), jnp.float32),
        pltpu.VMEM((CHUNK, D), jnp.float32),
        pltpu.VMEM((CHUNK,), jnp.float32),
    ],
)
def _decode_kernel(z_hbm, a_hbm, b_hbm, out_hbm, a_v, b_v, za_v, zb_v, dot_v):
    cid = lax.axis_index("c")
    sid = lax.axis_index("s")
    wid = sid * NC + cid

    def chunk_body(t, _):
        c = wid + t * (NC * NS)

        @pl.when(c < L_CHUNKS)
        def _():
            pltpu.sync_copy(a_hbm.at[pl.ds(c * CHUNK, CHUNK)], a_v)
            pltpu.sync_copy(b_hbm.at[pl.ds(c * CHUNK, CHUNK)], b_v)
            pltpu.sync_copy(z_hbm.at[a_v], za_v)
            pltpu.sync_copy(z_hbm.at[b_v], zb_v)

            def row_body(r, _):
                acc = za_v[r, pl.ds(0, LANES)] * zb_v[r, pl.ds(0, LANES)]
                for k in range(1, D // LANES):
                    acc = acc + (
                        za_v[r, pl.ds(k * LANES, LANES)]
                        * zb_v[r, pl.ds(k * LANES, LANES)]
                    )
                dot_v[r] = jnp.sum(acc)
                return 0

            lax.fori_loop(0, CHUNK, row_body, 0)
            pltpu.sync_copy(dot_v, out_hbm.at[pl.ds(c * CHUNK, CHUNK)])

        return 0

    lax.fori_loop(0, (L_CHUNKS + NC * NS - 1) // (NC * NS), chunk_body, 0)


RB = 1000
GRID = N // RB

_row = pl.BlockSpec((RB, D), lambda i: (i, 0))
_col = pl.BlockSpec((RB, 1), lambda i: (i, 0))
_wgt = pl.BlockSpec((D, D), lambda i: (0, 0))
_bias = pl.BlockSpec((1, D), lambda i: (0, 0))


def _tc_a_body(x_ref, dinv_ref, w1_ref, wf1_ref, bf1_ref, wf2_ref, bf2_ref,
               g1_ref, xf_ref):
    xb = x_ref[...]
    dinv = dinv_ref[...]
    g1_ref[...] = jnp.dot(xb, w1_ref[...], preferred_element_type=jnp.float32) * dinv
    t = jnp.maximum(
        jnp.dot(xb, wf1_ref[...], preferred_element_type=jnp.float32) + bf1_ref[...],
        0.0,
    )
    xf_ref[...] = jnp.dot(t, wf2_ref[...], preferred_element_type=jnp.float32) + bf2_ref[...]


_tc_a = pl.pallas_call(
    _tc_a_body,
    grid=(GRID,),
    in_specs=[_row, _col, _wgt, _wgt, _bias, _wgt, _bias],
    out_specs=[_row, _row],
    out_shape=[
        jax.ShapeDtypeStruct((N, D), jnp.float32),
        jax.ShapeDtypeStruct((N, D), jnp.float32),
    ],
)


def _tc_b_body(s0_ref, s1_ref, g1_ref, dinv_ref, b1_ref, w2_ref, g2_ref):
    dinv = dinv_ref[...]
    pre = (s0_ref[...] + s1_ref[...] - g1_ref[...]) * dinv + b1_ref[...]
    out1 = jnp.maximum(pre, 0.0)
    g2_ref[...] = jnp.dot(out1, w2_ref[...], preferred_element_type=jnp.float32) * dinv


_tc_b = pl.pallas_call(
    _tc_b_body,
    grid=(GRID,),
    in_specs=[_row, _row, _row, _col, _bias, _wgt],
    out_specs=_row,
    out_shape=jax.ShapeDtypeStruct((N, D), jnp.float32),
)


def _tc_c_body(s0_ref, s1_ref, g2_ref, dinv_ref, b2_ref, xf_ref, z_ref):
    pre = (s0_ref[...] + s1_ref[...] - g2_ref[...]) * dinv_ref[...] + b2_ref[...]
    z_ref[...] = 0.5 * pre + 0.5 * xf_ref[...]


_tc_c = pl.pallas_call(
    _tc_c_body,
    grid=(GRID,),
    in_specs=[_row, _row, _row, _col, _bias, _row],
    out_specs=_row,
    out_shape=jax.ShapeDtypeStruct((N, D), jnp.float32),
)


def kernel(x, edge_index, edge_label_index, W1, b1, W2, b2, Wf1, bf1, Wf2, bf2):
    ei = edge_index.astype(jnp.int32)
    eli = edge_label_index.astype(jnp.int32)
    src = ei[0]
    dst = ei[1]

    dinv_pad = _deg_kernel(dst)
    dinv = dinv_pad[:N].reshape(N, 1)

    b1r = b1.reshape(1, D)
    b2r = b2.reshape(1, D)
    bf1r = bf1.reshape(1, D)
    bf2r = bf2.reshape(1, D)

    g1, xf = _tc_a(x, dinv, W1, Wf1, bf1r, Wf2, bf2r)
    s1 = _agg_kernel(g1, src, dst)
    g2 = _tc_b(s1[0], s1[1], g1, dinv, b1r, W2)
    s2 = _agg_kernel(g2, src, dst)
    z = _tc_c(s2[0], s2[1], g2, dinv, b2r, xf)
    out = _decode_kernel(z, eli[0], eli[1])
    return out


# speedup vs baseline: 12.1981x; 12.1981x over previous
import functools

import jax
import jax.numpy as jnp
from jax import lax
from jax.experimental import pallas as pl
from jax.experimental.pallas import tpu as pltpu
from jax.experimental.pallas import tpu_sc as plsc

N = 10000
D = 128
E = 320000
LBL = 200000
NPAD = 10240

NC = 2
NS = 16
LANES = 16

CHUNK = 400
E_CHUNKS = E // CHUNK
E_CHUNKS_PER_TILE = E_CHUNKS // (NC * NS)
L_CHUNKS = LBL // CHUNK

_mesh = plsc.VectorSubcoreMesh(core_axis_name="c", subcore_axis_name="s")


def _fill(ref, start, count, value):
    v = jnp.full((LANES,), value, ref.dtype)

    def body(i, _):
        ref[pl.ds(start + i * LANES, LANES)] = v
        return 0

    lax.fori_loop(0, count // LANES, body, 0)


@functools.partial(
    pl.kernel,
    out_type=jax.ShapeDtypeStruct((NPAD,), jnp.float32),
    mesh=_mesh,
    scratch_types=[
        pltpu.VMEM((CHUNK,), jnp.int32),
        pltpu.VMEM((CHUNK,), jnp.float32),
        pltpu.VMEM((NPAD // NS,), jnp.float32),
        pltpu.VMEM_SHARED((NPAD,), jnp.float32),
    ],
)
def _deg_kernel(dst_hbm, dinv_hbm, idx_v, ones_v, slice_v, deg_sh):
    cid = lax.axis_index("c")
    sid = lax.axis_index("s")
    per = NPAD // NS

    _fill(slice_v, 0, per, 0.0)
    pltpu.sync_copy(slice_v, deg_sh.at[pl.ds(sid * per, per)])
    _fill(ones_v, 0, CHUNK, 1.0)
    plsc.subcore_barrier()

    def chunk_body(t, _):
        c = sid * (E_CHUNKS // NS) + t
        pltpu.sync_copy(dst_hbm.at[pl.ds(c * CHUNK, CHUNK)], idx_v)
        pltpu.sync_copy(ones_v, deg_sh.at[idx_v], add=True)
        return 0

    lax.fori_loop(0, E_CHUNKS // NS, chunk_body, 0)
    plsc.subcore_barrier()

    @pl.when(cid == 0)
    def _():
        pltpu.sync_copy(deg_sh.at[pl.ds(sid * per, per)],
                        dinv_hbm.at[pl.ds(sid * per, per)])


DH = D // 2

@functools.partial(
    pl.kernel,
    out_type=jax.ShapeDtypeStruct((NC, NPAD, DH), jnp.float32),
    mesh=_mesh,
    scratch_types=[
        pltpu.VMEM((CHUNK,), jnp.int32),
        pltpu.VMEM((CHUNK,), jnp.int32),
        pltpu.VMEM((CHUNK, DH), jnp.float32),
        pltpu.VMEM_SHARED((NPAD, DH), jnp.float32),
    ],
    compiler_params=pltpu.CompilerParams(use_tc_tiling_on_sc=False, needs_layout_passes=False),
)
def _agg_kernel(glo_hbm, ghi_hbm, src_hbm, dst_hbm, out_hbm,
                src_v, dst_v, rows_v, acc_sh):
    cid = lax.axis_index("c")
    sid = lax.axis_index("s")
    rpt = NPAD // NS

    @pl.when(cid == 0)
    def _():
        pltpu.sync_copy(glo_hbm.at[pl.ds(sid * rpt, rpt)],
                        acc_sh.at[pl.ds(sid * rpt, rpt)])

    @pl.when(cid == 1)
    def _():
        pltpu.sync_copy(ghi_hbm.at[pl.ds(sid * rpt, rpt)],
                        acc_sh.at[pl.ds(sid * rpt, rpt)])

    plsc.subcore_barrier()

    cpt = E_CHUNKS // NS

    def chunk_body(t, _):
        c = sid * cpt + t
        pltpu.sync_copy(src_hbm.at[pl.ds(c * CHUNK, CHUNK)], src_v)
        pltpu.sync_copy(dst_hbm.at[pl.ds(c * CHUNK, CHUNK)], dst_v)

        @pl.when(cid == 0)
        def _():
            pltpu.sync_copy(glo_hbm.at[src_v], rows_v)

        @pl.when(cid == 1)
        def _():
            pltpu.sync_copy(ghi_hbm.at[src_v], rows_v)

        pltpu.sync_copy(rows_v, acc_sh.at[dst_v], add=True)
        return 0

    lax.fori_loop(0, cpt, chunk_body, 0)
    plsc.subcore_barrier()

    pltpu.sync_copy(
        acc_sh.at[pl.ds(sid * rpt, rpt)],
        out_hbm.at[cid, pl.ds(sid * rpt, rpt)],
    )


@functools.partial(
    pl.kernel,
    out_type=jax.ShapeDtypeStruct((LBL,), jnp.float32),
    mesh=_mesh,
    scratch_types=[
        pltpu.VMEM((CHUNK,), jnp.int32),
        pltpu.VMEM((CHUNK,), jnp.int32),
        pltpu.VMEM((CHUNK, D), jnp.float32),
        pltpu.VMEM((CHUNK, D), jnp.float32),
        pltpu.VMEM((CHUNK,), jnp.float32),
    ],
    compiler_params=pltpu.CompilerParams(needs_layout_passes=False),
)
def _decode_kernel(z_hbm, a_hbm, b_hbm, out_hbm, a_v, b_v, za_v, zb_v, dot_v):
    cid = lax.axis_index("c")
    sid = lax.axis_index("s")
    wid = sid * NC + cid

    def chunk_body(t, _):
        c = wid + t * (NC * NS)

        @pl.when(c < L_CHUNKS)
        def _():
            pltpu.sync_copy(a_hbm.at[pl.ds(c * CHUNK, CHUNK)], a_v)
            pltpu.sync_copy(b_hbm.at[pl.ds(c * CHUNK, CHUNK)], b_v)
            pltpu.sync_copy(z_hbm.at[a_v], za_v)
            pltpu.sync_copy(z_hbm.at[b_v], zb_v)

            lane = lax.iota(jnp.int32, LANES)

            def group_body(gidx, _):
                base = gidx * LANES
                vec = jnp.zeros((LANES,), jnp.float32)
                for j in range(LANES):
                    r = base + j
                    acc = za_v[r, pl.ds(0, LANES)] * zb_v[r, pl.ds(0, LANES)]
                    for k in range(1, D // LANES):
                        acc = acc + (
                            za_v[r, pl.ds(k * LANES, LANES)]
                            * zb_v[r, pl.ds(k * LANES, LANES)]
                        )
                    vec = jnp.where(lane == j, jnp.sum(acc), vec)
                dot_v[pl.ds(base, LANES)] = vec
                return 0

            lax.fori_loop(0, CHUNK // LANES, group_body, 0)
            pltpu.sync_copy(dot_v, out_hbm.at[pl.ds(c * CHUNK, CHUNK)])

        return 0

    lax.fori_loop(0, (L_CHUNKS + NC * NS - 1) // (NC * NS), chunk_body, 0)


RB = 1280
GRID = NPAD // RB

_row = pl.BlockSpec((RB, D), lambda i: (i, 0))
_col = pl.BlockSpec((RB, 1), lambda i: (i, 0))
_wgt = pl.BlockSpec((D, D), lambda i: (0, 0))
_bias = pl.BlockSpec((1, D), lambda i: (0, 0))


def _tc_a_body(x_ref, deg_ref, w1_ref, wf1_ref, bf1_ref, wf2_ref, bf2_ref,
               g1_ref, xf_ref, dinv_ref):
    xb = x_ref[...]
    dinv = lax.rsqrt(deg_ref[...] + 1.0)
    dinv_ref[...] = dinv
    g1_ref[...] = jnp.dot(xb, w1_ref[...], preferred_element_type=jnp.float32) * dinv
    t = jnp.maximum(
        jnp.dot(xb, wf1_ref[...], preferred_element_type=jnp.float32) + bf1_ref[...],
        0.0,
    )
    xf_ref[...] = jnp.dot(t, wf2_ref[...], preferred_element_type=jnp.float32) + bf2_ref[...]


_tc_a = pl.pallas_call(
    _tc_a_body,
    grid=(GRID,),
    in_specs=[_row, _col, _wgt, _wgt, _bias, _wgt, _bias],
    out_specs=[_row, _row, _col],
    out_shape=[
        jax.ShapeDtypeStruct((NPAD, D), jnp.float32),
        jax.ShapeDtypeStruct((NPAD, D), jnp.float32),
        jax.ShapeDtypeStruct((NPAD, 1), jnp.float32),
    ],
)


def _tc_b_body(s_ref, dinv_ref, b1_ref, w2_ref, g2_ref):
    dinv = dinv_ref[...]
    pre = s_ref[...] * dinv + b1_ref[...]
    out1 = jnp.maximum(pre, 0.0)
    g2_ref[...] = jnp.dot(out1, w2_ref[...], preferred_element_type=jnp.float32) * dinv


_tc_b = pl.pallas_call(
    _tc_b_body,
    grid=(GRID,),
    in_specs=[_row, _col, _bias, _wgt],
    out_specs=_row,
    out_shape=jax.ShapeDtypeStruct((NPAD, D), jnp.float32),
)


def _tc_c_body(s_ref, dinv_ref, b2_ref, xf_ref, z_ref):
    pre = s_ref[...] * dinv_ref[...] + b2_ref[...]
    z_ref[...] = 0.5 * pre + 0.5 * xf_ref[...]


_tc_c = pl.pallas_call(
    _tc_c_body,
    grid=(GRID,),
    in_specs=[_row, _col, _bias, _row],
    out_specs=_row,
    out_shape=jax.ShapeDtypeStruct((NPAD, D), jnp.float32),
)


def kernel(x, edge_index, edge_label_index, W1, b1, W2, b2, Wf1, bf1, Wf2, bf2):
    ei = edge_index.astype(jnp.int32)
    eli = edge_label_index.astype(jnp.int32)
    src = ei[0]
    dst = ei[1]

    xp = jnp.pad(x, ((0, NPAD - N), (0, 0)))
    deg = _deg_kernel(dst).reshape(NPAD, 1)

    b1r = b1.reshape(1, D)
    b2r = b2.reshape(1, D)
    bf1r = bf1.reshape(1, D)
    bf2r = bf2.reshape(1, D)

    g1, xf, dinv = _tc_a(xp, deg, W1, Wf1, bf1r, Wf2, bf2r)
    s1h = _agg_kernel(g1[:, :DH], g1[:, DH:], src, dst)
    s1 = s1h.transpose(1, 0, 2).reshape(NPAD, D)
    g2 = _tc_b(s1, dinv, b1r, W2)
    s2h = _agg_kernel(g2[:, :DH], g2[:, DH:], src, dst)
    s2 = s2h.transpose(1, 0, 2).reshape(NPAD, D)
    z = _tc_c(s2, dinv, b2r, xf)
    out = _decode_kernel(z, eli[0], eli[1])
    return out


# speedup vs baseline: 15.5253x; 1.2728x over previous
import functools

import jax
import jax.numpy as jnp
from jax import lax
from jax.experimental import pallas as pl
from jax.experimental.pallas import tpu as pltpu
from jax.experimental.pallas import tpu_sc as plsc

N = 10000
D = 128
E = 320000
LBL = 200000
NPAD = 10240

NC = 2
NS = 16
LANES = 16

CHUNK = 400
E_CHUNKS = E // CHUNK
E_CHUNKS_PER_TILE = E_CHUNKS // (NC * NS)
L_CHUNKS = LBL // CHUNK

_mesh = plsc.VectorSubcoreMesh(core_axis_name="c", subcore_axis_name="s")


def _fill(ref, start, count, value):
    v = jnp.full((LANES,), value, ref.dtype)

    def body(i, _):
        ref[pl.ds(start + i * LANES, LANES)] = v
        return 0

    lax.fori_loop(0, count // LANES, body, 0)


@functools.partial(
    pl.kernel,
    out_type=jax.ShapeDtypeStruct((NPAD,), jnp.float32),
    mesh=_mesh,
    scratch_types=[
        pltpu.VMEM((CHUNK,), jnp.int32),
        pltpu.VMEM((CHUNK,), jnp.float32),
        pltpu.VMEM((NPAD // NS,), jnp.float32),
        pltpu.VMEM_SHARED((NPAD,), jnp.float32),
    ],
)
def _deg_kernel(dst_hbm, dinv_hbm, idx_v, ones_v, slice_v, deg_sh):
    cid = lax.axis_index("c")
    sid = lax.axis_index("s")
    per = NPAD // NS

    _fill(slice_v, 0, per, 0.0)
    pltpu.sync_copy(slice_v, deg_sh.at[pl.ds(sid * per, per)])
    _fill(ones_v, 0, CHUNK, 1.0)
    plsc.subcore_barrier()

    def chunk_body(t, _):
        c = sid * (E_CHUNKS // NS) + t
        pltpu.sync_copy(dst_hbm.at[pl.ds(c * CHUNK, CHUNK)], idx_v)
        pltpu.sync_copy(ones_v, deg_sh.at[idx_v], add=True)
        return 0

    lax.fori_loop(0, E_CHUNKS // NS, chunk_body, 0)
    plsc.subcore_barrier()

    @pl.when(cid == 0)
    def _():
        pltpu.sync_copy(deg_sh.at[pl.ds(sid * per, per)],
                        dinv_hbm.at[pl.ds(sid * per, per)])


DH = D // 2
ECHUNK = 400
E_CHUNKS2 = E // ECHUNK
CPT = E_CHUNKS2 // NS


@functools.partial(
    pl.kernel,
    out_type=jax.ShapeDtypeStruct((NC, NPAD, DH), jnp.float32),
    mesh=_mesh,
    scratch_types=[
        pltpu.VMEM((ECHUNK,), jnp.int32),
        pltpu.VMEM((ECHUNK,), jnp.int32),
        pltpu.VMEM((ECHUNK,), jnp.int32),
        pltpu.VMEM((ECHUNK,), jnp.int32),
        pltpu.VMEM((ECHUNK, DH), jnp.float32),
        pltpu.VMEM((ECHUNK, DH), jnp.float32),
        pltpu.SemaphoreType.DMA,
        pltpu.SemaphoreType.DMA,
        pltpu.VMEM_SHARED((NPAD, DH), jnp.float32),
    ],
    compiler_params=pltpu.CompilerParams(use_tc_tiling_on_sc=False, needs_layout_passes=False),
)
def _agg_kernel(gflat_hbm, src2_hbm, dst_hbm, out_hbm,
                src_a, src_b, dst_a, dst_b, rows_a, rows_b, sem_a, sem_b,
                acc_sh):
    cid = lax.axis_index("c")
    sid = lax.axis_index("s")
    rpt = NPAD // NS

    pltpu.sync_copy(gflat_hbm.at[pl.ds(cid * NPAD + sid * rpt, rpt)],
                    acc_sh.at[pl.ds(sid * rpt, rpt)])
    plsc.subcore_barrier()

    bufs = [(src_a, dst_a, rows_a, sem_a), (src_b, dst_b, rows_b, sem_b)]

    def issue(t, buf):
        sv, dv, rv, sem = buf
        c = sid * CPT + t
        pltpu.sync_copy(src2_hbm.at[cid, pl.ds(c * ECHUNK, ECHUNK)], sv)
        pltpu.sync_copy(dst_hbm.at[pl.ds(c * ECHUNK, ECHUNK)], dv)
        pltpu.async_copy(gflat_hbm.at[sv], rv, sem)

    issue(0, bufs[0])
    for t in range(CPT):
        cur = bufs[t % 2]
        if t + 1 < CPT:
            issue(t + 1, bufs[(t + 1) % 2])
        sv, dv, rv, sem = cur
        pltpu.make_async_copy(gflat_hbm.at[sv], rv, sem).wait()
        pltpu.sync_copy(rv, acc_sh.at[dv], add=True)

    plsc.subcore_barrier()
    pltpu.sync_copy(acc_sh.at[pl.ds(sid * rpt, rpt)],
                    out_hbm.at[cid, pl.ds(sid * rpt, rpt)])


LCHUNK = 200
L_CHUNKS2 = LBL // LCHUNK
L_ITER = (L_CHUNKS2 + NC * NS - 1) // (NC * NS)


@functools.partial(
    pl.kernel,
    out_type=jax.ShapeDtypeStruct((LBL,), jnp.float32),
    mesh=_mesh,
    scratch_types=[
        pltpu.VMEM((LCHUNK,), jnp.int32),
        pltpu.VMEM((LCHUNK,), jnp.int32),
        pltpu.VMEM((LCHUNK,), jnp.int32),
        pltpu.VMEM((LCHUNK,), jnp.int32),
        pltpu.VMEM((LCHUNK, D), jnp.float32),
        pltpu.VMEM((LCHUNK, D), jnp.float32),
        pltpu.VMEM((LCHUNK, D), jnp.float32),
        pltpu.VMEM((LCHUNK, D), jnp.float32),
        pltpu.VMEM((LCHUNK,), jnp.float32),
        pltpu.SemaphoreType.DMA,
        pltpu.SemaphoreType.DMA,
    ],
    compiler_params=pltpu.CompilerParams(needs_layout_passes=False),
)
def _decode_kernel(z_hbm, a_hbm, b_hbm, out_hbm,
                   a_va, b_va, a_vb, b_vb, za_va, zb_va, za_vb, zb_vb,
                   dot_v, sem_a, sem_b):
    cid = lax.axis_index("c")
    sid = lax.axis_index("s")
    wid = sid * NC + cid

    bufs = [(a_va, b_va, za_va, zb_va, sem_a), (a_vb, b_vb, za_vb, zb_vb, sem_b)]
    lane = lax.iota(jnp.int32, LANES)

    def issue(t, buf):
        av, bv, zav, zbv, sem = buf
        c = wid + t * (NC * NS)

        @pl.when(c < L_CHUNKS2)
        def _():
            pltpu.sync_copy(a_hbm.at[pl.ds(c * LCHUNK, LCHUNK)], av)
            pltpu.sync_copy(b_hbm.at[pl.ds(c * LCHUNK, LCHUNK)], bv)
            pltpu.async_copy(z_hbm.at[av], zav, sem)
            pltpu.async_copy(z_hbm.at[bv], zbv, sem)

    def process(t, buf):
        av, bv, zav, zbv, sem = buf
        c = wid + t * (NC * NS)

        @pl.when(c < L_CHUNKS2)
        def _():
            pltpu.make_async_copy(z_hbm.at[av], zav, sem).wait()
            pltpu.make_async_copy(z_hbm.at[bv], zbv, sem).wait()

            def group_body(gidx, _):
                base = gidx * LANES
                vec = jnp.zeros((LANES,), jnp.float32)
                for j in range(LANES):
                    r = base + j
                    acc = zav[r, pl.ds(0, LANES)] * zbv[r, pl.ds(0, LANES)]
                    for k in range(1, D // LANES):
                        acc = acc + (
                            zav[r, pl.ds(k * LANES, LANES)]
                            * zbv[r, pl.ds(k * LANES, LANES)]
                        )
                    vec = jnp.where(lane == j, jnp.sum(acc), vec)
                dot_v[pl.ds(base, LANES)] = vec
                return 0

            lax.fori_loop(0, LCHUNK // LANES, group_body, 0)
            pltpu.sync_copy(dot_v, out_hbm.at[pl.ds(c * LCHUNK, LCHUNK)])

    issue(0, bufs[0])

    def chunk_iter(t, _):
        @pl.when(t % 2 == 0)
        def _():
            @pl.when(t + 1 < L_ITER)
            def _():
                issue(t + 1, bufs[1])
            process(t, bufs[0])

        @pl.when(t % 2 == 1)
        def _():
            @pl.when(t + 1 < L_ITER)
            def _():
                issue(t + 1, bufs[0])
            process(t, bufs[1])

        return 0

    lax.fori_loop(0, L_ITER, chunk_iter, 0)


RB = 1280
GRID = NPAD // RB

_row = pl.BlockSpec((RB, D), lambda i: (i, 0))
_col = pl.BlockSpec((RB, 1), lambda i: (i, 0))
_wgt = pl.BlockSpec((D, D), lambda i: (0, 0))
_bias = pl.BlockSpec((1, D), lambda i: (0, 0))


def _tc_a_body(x_ref, deg_ref, w1_ref, wf1_ref, bf1_ref, wf2_ref, bf2_ref,
               g1_ref, xf_ref, dinv_ref):
    xb = x_ref[...]
    dinv = lax.rsqrt(deg_ref[...] + 1.0)
    dinv_ref[...] = dinv
    g1_ref[...] = jnp.dot(xb, w1_ref[...], preferred_element_type=jnp.float32) * dinv
    t = jnp.maximum(
        jnp.dot(xb, wf1_ref[...], preferred_element_type=jnp.float32) + bf1_ref[...],
        0.0,
    )
    xf_ref[...] = jnp.dot(t, wf2_ref[...], preferred_element_type=jnp.float32) + bf2_ref[...]


_tc_a = pl.pallas_call(
    _tc_a_body,
    grid=(GRID,),
    in_specs=[_row, _col, _wgt, _wgt, _bias, _wgt, _bias],
    out_specs=[_row, _row, _col],
    out_shape=[
        jax.ShapeDtypeStruct((NPAD, D), jnp.float32),
        jax.ShapeDtypeStruct((NPAD, D), jnp.float32),
        jax.ShapeDtypeStruct((NPAD, 1), jnp.float32),
    ],
)


def _tc_b_body(s_ref, dinv_ref, b1_ref, w2_ref, g2_ref):
    dinv = dinv_ref[...]
    pre = s_ref[...] * dinv + b1_ref[...]
    out1 = jnp.maximum(pre, 0.0)
    g2_ref[...] = jnp.dot(out1, w2_ref[...], preferred_element_type=jnp.float32) * dinv


_tc_b = pl.pallas_call(
    _tc_b_body,
    grid=(GRID,),
    in_specs=[_row, _col, _bias, _wgt],
    out_specs=_row,
    out_shape=jax.ShapeDtypeStruct((NPAD, D), jnp.float32),
)


def _tc_c_body(s_ref, dinv_ref, b2_ref, xf_ref, z_ref):
    pre = s_ref[...] * dinv_ref[...] + b2_ref[...]
    z_ref[...] = 0.5 * pre + 0.5 * xf_ref[...]


_tc_c = pl.pallas_call(
    _tc_c_body,
    grid=(GRID,),
    in_specs=[_row, _col, _bias, _row],
    out_specs=_row,
    out_shape=jax.ShapeDtypeStruct((NPAD, D), jnp.float32),
)


def kernel(x, edge_index, edge_label_index, W1, b1, W2, b2, Wf1, bf1, Wf2, bf2):
    ei = edge_index.astype(jnp.int32)
    eli = edge_label_index.astype(jnp.int32)
    src = ei[0]
    dst = ei[1]

    xp = jnp.pad(x, ((0, NPAD - N), (0, 0)))
    deg = _deg_kernel(dst).reshape(NPAD, 1)

    b1r = b1.reshape(1, D)
    b2r = b2.reshape(1, D)
    bf1r = bf1.reshape(1, D)
    bf2r = bf2.reshape(1, D)

    src2 = jnp.stack([src, src + NPAD])

    g1, xf, dinv = _tc_a(xp, deg, W1, Wf1, bf1r, Wf2, bf2r)
    g1f = jnp.concatenate([g1[:, :DH], g1[:, DH:]], axis=0)
    s1h = _agg_kernel(g1f, src2, dst)
    s1 = s1h.transpose(1, 0, 2).reshape(NPAD, D)
    g2 = _tc_b(s1, dinv, b1r, W2)
    g2f = jnp.concatenate([g2[:, :DH], g2[:, DH:]], axis=0)
    s2h = _agg_kernel(g2f, src2, dst)
    s2 = s2h.transpose(1, 0, 2).reshape(NPAD, D)
    z = _tc_c(s2, dinv, b2r, xf)
    out = _decode_kernel(z, eli[0], eli[1])
    return out
